# baseline (speedup 1.0000x reference)
import functools

import jax
import jax.numpy as jnp
from jax import lax
from jax.experimental import pallas as pl
from jax.experimental.pallas import tpu as pltpu
from jax.experimental.pallas import tpu_sc as plsc

_L = 16


def _make_sc_kernels(n_nodes, n_edges, nc, ns):
    nw = nc * ns
    nchunk = n_edges // 128
    wch = -(-nchunk // nw) + 1
    zvec = n_nodes // _L

    mesh = plsc.VectorSubcoreMesh(core_axis_name="c", subcore_axis_name="s")
    out_t = jax.ShapeDtypeStruct((nw, n_nodes), jnp.float32)
    cparams = pltpu.CompilerParams(needs_layout_passes=False)

    def chunk_bounds(wid):
        start = (nchunk * wid) // nw
        end = (nchunk * (wid + 1)) // nw
        base = jnp.minimum(start, nchunk - wch)
        return start, end, base

    def zero_acc(acc_v):
        zero = jnp.zeros((_L,), jnp.float32)

        def zf(k, _):
            for j in range(8):
                acc_v[pl.ds((k * 8 + j) * _L, _L)] = zero
            return 0
        lax.fori_loop(0, zvec // 8, zf, 0)
        for j in range(zvec - (zvec // 8) * 8):
            acc_v[pl.ds(((zvec // 8) * 8 + j) * _L, _L)] = zero

    @functools.partial(
        pl.kernel,
        out_type=out_t,
        mesh=mesh,
        scratch_types=[
            pltpu.VMEM((2, wch * 128), jnp.int32),
            pltpu.VMEM((n_nodes,), jnp.float32),
            pltpu.SemaphoreType.DMA,
        ],
        compiler_params=cparams,
    )
    def deg_kernel(edge_hbm, out_hbm, ed_v, acc_v, sem):
        c = lax.axis_index("c")
        s = lax.axis_index("s")
        wid = s * nc + c
        start, end, base = chunk_bounds(wid)
        cp = pltpu.async_copy(
            edge_hbm.at[:, pl.ds(base * 128, wch * 128)], ed_v, sem)
        zero_acc(acc_v)
        cp.wait()
        ones = jnp.ones((_L,), jnp.float32)
        off0 = (start - base) * 128

        @plsc.parallel_loop(0, (end - start) * 128, _L, unroll=16)
        def scat(k):
            iv = ed_v[1, pl.ds(off0 + k, _L)]
            plsc.addupdate_scatter(acc_v, [iv], ones)
        pltpu.sync_copy(acc_v, out_hbm.at[wid])

    @functools.partial(
        pl.kernel,
        out_type=out_t,
        mesh=mesh,
        scratch_types=[
            pltpu.VMEM((2, wch * 128), jnp.int32),
            pltpu.VMEM((n_nodes,), jnp.float32),
            pltpu.VMEM((n_nodes,), jnp.float32),
            pltpu.SemaphoreType.DMA,
        ],
        compiler_params=cparams,
    )
    def edge_kernel(edge_hbm, t_hbm, out_hbm, ed_v, t_v, acc_v, sem):
        c = lax.axis_index("c")
        s = lax.axis_index("s")
        wid = s * nc + c
        start, end, base = chunk_bounds(wid)
        cp1 = pltpu.async_copy(
            edge_hbm.at[:, pl.ds(base * 128, wch * 128)], ed_v, sem)
        cp2 = pltpu.async_copy(t_hbm, t_v, sem)
        zero_acc(acc_v)
        cp1.wait()
        cp2.wait()
        off0 = (start - base) * 128

        @plsc.parallel_loop(0, (end - start) * 128, _L, unroll=16)
        def body(k):
            sv = ed_v[0, pl.ds(off0 + k, _L)]
            dv = ed_v[1, pl.ds(off0 + k, _L)]
            vals = plsc.load_gather(t_v, [sv])
            plsc.addupdate_scatter(acc_v, [dv], vals)
        pltpu.sync_copy(acc_v, out_hbm.at[wid])

    return deg_kernel, edge_kernel


def _s_body(state_ref, w1_ref, w2_ref, s_ref):
    w_row = lax.dot_general(w2_ref[...], w1_ref[...],
                            (((0,), (1,)), ((), ())),
                            preferred_element_type=jnp.float32)
    s_row = lax.dot_general(w_row, state_ref[...],
                            (((1,), (1,)), ((), ())),
                            preferred_element_type=jnp.float32)
    s_ref[...] = s_row[0]


def _t_body(s_ref, cnt_ref, t_ref, dinv_ref):
    cnt = jnp.sum(cnt_ref[...], axis=0)
    dinv = lax.rsqrt(cnt + 1.0)
    t_ref[...] = dinv * s_ref[...]
    dinv_ref[...] = dinv


def _final_body(acc_ref, t_ref, dinv_ref, b1_ref, w2_ref, b2_ref, out_ref):
    acc = jnp.sum(acc_ref[...], axis=0)
    cconst = jnp.sum(b1_ref[...] * w2_ref[...]) + jnp.sum(b2_ref[...])
    out_ref[...] = dinv_ref[...] * (acc + t_ref[...]) + cconst


def kernel(state, edge_index, W1, b1, W2, b2):
    n_nodes, _ = state.shape
    n_edges = edge_index.shape[1]
    info = plsc.get_sparse_core_info()
    nc, ns = info.num_cores, info.num_subcores

    deg_k, edge_k = _make_sc_kernels(n_nodes, n_edges, nc, ns)

    cnt32 = deg_k(edge_index)

    s = pl.pallas_call(
        _s_body,
        out_shape=jax.ShapeDtypeStruct((n_nodes,), jnp.float32),
    )(state, W1, W2)

    t, dinv = pl.pallas_call(
        _t_body,
        out_shape=[jax.ShapeDtypeStruct((n_nodes,), jnp.float32)] * 2,
    )(s, cnt32)

    acc32 = edge_k(edge_index, t)

    out = pl.pallas_call(
        _final_body,
        out_shape=jax.ShapeDtypeStruct((n_nodes,), jnp.float32),
    )(acc32, t, dinv, b1, W2[:, 0], b2)
    return out.reshape(n_nodes, 1)


# speedup vs baseline: 1.0064x; 1.0064x over previous
import functools

import jax
import jax.numpy as jnp
from jax import lax
from jax.experimental import pallas as pl
from jax.experimental.pallas import tpu as pltpu
from jax.experimental.pallas import tpu_sc as plsc

_L = 16


def _make_sc_kernels(n_nodes, n_edges, nc, ns):
    nw = nc * ns
    nchunk = n_edges // 128
    wch = -(-nchunk // nw) + 1
    zvec = n_nodes // _L

    mesh = plsc.VectorSubcoreMesh(core_axis_name="c", subcore_axis_name="s")
    out_t = jax.ShapeDtypeStruct((nw, n_nodes), jnp.float32)
    cparams = pltpu.CompilerParams(needs_layout_passes=False)

    def chunk_bounds(wid):
        start = (nchunk * wid) // nw
        end = (nchunk * (wid + 1)) // nw
        base = jnp.minimum(start, nchunk - wch)
        return start, end, base

    def zero_acc(acc_v):
        zero = jnp.zeros((_L,), jnp.float32)

        def zf(k, _):
            for j in range(8):
                acc_v[pl.ds((k * 8 + j) * _L, _L)] = zero
            return 0
        lax.fori_loop(0, zvec // 8, zf, 0)
        for j in range(zvec - (zvec // 8) * 8):
            acc_v[pl.ds(((zvec // 8) * 8 + j) * _L, _L)] = zero

    @functools.partial(
        pl.kernel,
        out_type=out_t,
        mesh=mesh,
        scratch_types=[
            pltpu.VMEM((2, wch * 128), jnp.int32),
            pltpu.VMEM((n_nodes,), jnp.float32),
            pltpu.SemaphoreType.DMA,
        ],
        compiler_params=cparams,
    )
    def deg_kernel(edge_hbm, out_hbm, ed_v, acc_v, sem):
        c = lax.axis_index("c")
        s = lax.axis_index("s")
        wid = s * nc + c
        start, end, base = chunk_bounds(wid)
        cp = pltpu.async_copy(
            edge_hbm.at[:, pl.ds(base * 128, wch * 128)], ed_v, sem)
        zero_acc(acc_v)
        cp.wait()
        ones = jnp.ones((_L,), jnp.float32)
        off0 = (start - base) * 128

        @plsc.parallel_loop(0, (end - start) * 128, _L, unroll=8)
        def scat(k):
            iv = ed_v[1, pl.ds(off0 + k, _L)]
            plsc.addupdate_scatter(acc_v, [iv], ones)
        pltpu.sync_copy(acc_v, out_hbm.at[wid])

    @functools.partial(
        pl.kernel,
        out_type=out_t,
        mesh=mesh,
        scratch_types=[
            pltpu.VMEM((2, wch * 128), jnp.int32),
            pltpu.VMEM((n_nodes,), jnp.float32),
            pltpu.VMEM((n_nodes,), jnp.float32),
            pltpu.SemaphoreType.DMA,
        ],
        compiler_params=cparams,
    )
    def edge_kernel(edge_hbm, t_hbm, out_hbm, ed_v, t_v, acc_v, sem):
        c = lax.axis_index("c")
        s = lax.axis_index("s")
        wid = s * nc + c
        start, end, base = chunk_bounds(wid)
        cp1 = pltpu.async_copy(
            edge_hbm.at[:, pl.ds(base * 128, wch * 128)], ed_v, sem)
        cp2 = pltpu.async_copy(t_hbm, t_v, sem)
        zero_acc(acc_v)
        cp1.wait()
        cp2.wait()
        off0 = (start - base) * 128

        @plsc.parallel_loop(0, (end - start) * 128, _L, unroll=8)
        def body(k):
            sv = ed_v[0, pl.ds(off0 + k, _L)]
            dv = ed_v[1, pl.ds(off0 + k, _L)]
            vals = plsc.load_gather(t_v, [sv])
            plsc.addupdate_scatter(acc_v, [dv], vals)
        pltpu.sync_copy(acc_v, out_hbm.at[wid])

    return deg_kernel, edge_kernel


def _s_body(state_ref, w1_ref, w2_ref, s_ref):
    w_row = lax.dot_general(w2_ref[...], w1_ref[...],
                            (((0,), (1,)), ((), ())),
                            preferred_element_type=jnp.float32)
    s_row = lax.dot_general(w_row, state_ref[...],
                            (((1,), (1,)), ((), ())),
                            preferred_element_type=jnp.float32)
    s_ref[...] = s_row[0]


def _t_body(s_ref, cnt_ref, t_ref, dinv_ref):
    cnt = jnp.sum(cnt_ref[...], axis=0)
    dinv = lax.rsqrt(cnt + 1.0)
    t_ref[...] = dinv * s_ref[...]
    dinv_ref[...] = dinv


def _final_body(acc_ref, t_ref, dinv_ref, b1_ref, w2_ref, b2_ref, out_ref):
    acc = jnp.sum(acc_ref[...], axis=0)
    cconst = jnp.sum(b1_ref[...] * w2_ref[...]) + jnp.sum(b2_ref[...])
    out_ref[...] = dinv_ref[...] * (acc + t_ref[...]) + cconst


def kernel(state, edge_index, W1, b1, W2, b2):
    n_nodes, _ = state.shape
    n_edges = edge_index.shape[1]
    info = plsc.get_sparse_core_info()
    nc, ns = info.num_cores, info.num_subcores

    deg_k, edge_k = _make_sc_kernels(n_nodes, n_edges, nc, ns)

    cnt32 = deg_k(edge_index)

    s = pl.pallas_call(
        _s_body,
        out_shape=jax.ShapeDtypeStruct((n_nodes,), jnp.float32),
    )(state, W1, W2)

    t, dinv = pl.pallas_call(
        _t_body,
        out_shape=[jax.ShapeDtypeStruct((n_nodes,), jnp.float32)] * 2,
    )(s, cnt32)

    acc32 = edge_k(edge_index, t)

    out = pl.pallas_call(
        _final_body,
        out_shape=jax.ShapeDtypeStruct((n_nodes,), jnp.float32),
    )(acc32, t, dinv, b1, W2[:, 0], b2)
    return out.reshape(n_nodes, 1)
